# baseline (speedup 1.0000x reference)
import functools

import jax
import jax.numpy as jnp
from jax import lax
from jax.experimental import pallas as pl
from jax.experimental.pallas import tpu as pltpu
from jax.experimental.pallas import tpu_sc as plsc

M = 50000
E = 800000
H = 64
HH = 32
NC = 2
NS = 16
EC = 128
NR = E // EC
G8 = NR // 8
SS = 16
NR_PAD = 6304
M_T = 50048
RPT = M_T // NS
ZR = 136
RB = 2000


def _tc_gelu(x):
    return 0.5 * x * (1.0 + lax.erf(x * 0.7071067811865476))


def _edge_body(a_hbm, b_hbm, src_hbm, dsta_hbm, out_hbm,
               sidx, didxa, didxr, bufa, bufb, bufg, zb, tbl,
               sema0, sema1, semb0, semb1):
    c = lax.axis_index("c")
    s = lax.axis_index("s")
    sems = ((sema0, semb0), (sema1, semb1))
    zero16 = jnp.zeros((16,), jnp.float32)

    @pl.loop(0, ZR)
    def _zfill(r):
        zb[r, pl.ds(0, 16)] = zero16
        zb[r, pl.ds(16, 16)] = zero16

    base = s * RPT

    @pl.loop(0, RPT // ZR)
    def _zcopy(k):
        pltpu.sync_copy(zb, tbl.at[pl.ds(base + k * ZR, ZR)])

    plsc.subcore_barrier()

    start = 8 * ((G8 * s) // NS)
    end = jnp.where(s == NS - 1, NR, 8 * ((G8 * (s + 1)) // NS))
    cbase = c * NR_PAD

    def _issue(j, b):
        pltpu.async_copy(a_hbm.at[sidx.at[j]], bufa.at[b], sems[b][0])
        pltpu.async_copy(b_hbm.at[didxa.at[j]], bufb.at[b], sems[b][1])

    def _wait(j, b):
        pltpu.make_async_copy(a_hbm.at[sidx.at[j]], bufa.at[b],
                              sems[b][0]).wait()
        pltpu.make_async_copy(b_hbm.at[didxa.at[j]], bufb.at[b],
                              sems[b][1]).wait()

    def _compute(b):
        @pl.loop(0, EC, unroll=4)
        def _rows(r):
            for half in (0, 1):
                sl = pl.ds(16 * half, 16)
                x = bufa[b, r, sl] + bufb[b, r, sl]
                y = x + 0.044715 * (x * x * x)
                e = jnp.exp(-1.5957691216057308 * y)
                bufg[r, sl] = x / (1.0 + e)

    @pl.loop(start, end, step=SS)
    def _outer(g):
        g = pl.multiple_of(g, 8)
        m = jnp.minimum(end - g, SS)
        pltpu.sync_copy(src_hbm.at[pl.ds(cbase + g, SS)], sidx)
        pltpu.sync_copy(dsta_hbm.at[pl.ds(cbase + g, SS)], didxa)
        pltpu.sync_copy(dsta_hbm.at[pl.ds(g, SS)], didxr)
        _issue(0, 0)

        @pl.loop(0, (m + 1) // 2)
        def _pairs(p):
            j0 = 2 * p
            j1 = j0 + 1
            _wait(j0, 0)

            @pl.when(j1 < m)
            def _():
                _issue(j1, 1)

            _compute(0)
            pltpu.sync_copy(bufg, tbl.at[didxr.at[j0]], add=True)

            @pl.when(j1 < m)
            def _():
                _wait(j1, 1)

                @pl.when(j1 + 1 < m)
                def _():
                    _issue(j1 + 1, 0)

                _compute(1)
                pltpu.sync_copy(bufg, tbl.at[didxr.at[j1]], add=True)

    plsc.subcore_barrier()
    pltpu.sync_copy(tbl.at[pl.ds(base, RPT)],
                    out_hbm.at[pl.ds(c * M_T + base, RPT)])


@functools.cache
def _edge_call():
    return pl.kernel(
        _edge_body,
        out_type=jax.ShapeDtypeStruct((NC * M_T, HH), jnp.float32),
        mesh=plsc.VectorSubcoreMesh(core_axis_name="c", subcore_axis_name="s",
                                    num_cores=NC, num_subcores=NS),
        scratch_types=[
            pltpu.VMEM((SS, EC), jnp.int32),
            pltpu.VMEM((SS, EC), jnp.int32),
            pltpu.VMEM((SS, EC), jnp.int32),
            pltpu.VMEM((2, EC, HH), jnp.float32),
            pltpu.VMEM((2, EC, HH), jnp.float32),
            pltpu.VMEM((EC, HH), jnp.float32),
            pltpu.VMEM((ZR, HH), jnp.float32),
            pltpu.VMEM_SHARED((M_T, HH), jnp.float32),
            pltpu.SemaphoreType.DMA,
            pltpu.SemaphoreType.DMA,
            pltpu.SemaphoreType.DMA,
            pltpu.SemaphoreType.DMA,
        ],
        compiler_params=pltpu.CompilerParams(use_tc_tiling_on_sc=False),
    )


def _cnt_body(dsta_hbm, out_hbm, didx, ones, zb, ctbl):
    c = lax.axis_index("c")
    s = lax.axis_index("s")
    zero16 = jnp.zeros((16,), jnp.float32)
    onev = jnp.where(lax.broadcasted_iota(jnp.int32, (16,), 0) == 0,
                     1.0, 0.0).astype(jnp.float32)

    @pl.loop(0, EC)
    def _ofill(r):
        ones[r, pl.ds(0, 16)] = onev

    @pl.loop(0, ZR)
    def _zfill(r):
        zb[r, pl.ds(0, 16)] = zero16

    base = s * RPT

    @pl.loop(0, RPT // ZR)
    def _zcopy(k):
        pltpu.sync_copy(zb, ctbl.at[pl.ds(base + k * ZR, ZR)])

    plsc.subcore_barrier()

    w = c * NS + s
    nw = NC * NS
    start = 8 * ((G8 * w) // nw)
    end = jnp.where(w == nw - 1, NR, 8 * ((G8 * (w + 1)) // nw))

    @pl.loop(start, end, step=SS)
    def _outer(g):
        g = pl.multiple_of(g, 8)
        m = jnp.minimum(end - g, SS)
        pltpu.sync_copy(dsta_hbm.at[pl.ds(g, SS)], didx)

        @pl.loop(0, m)
        def _inner(j):
            pltpu.sync_copy(ones, ctbl.at[didx.at[j]], add=True)

    plsc.subcore_barrier()
    pltpu.sync_copy(ctbl.at[pl.ds(base, RPT)],
                    out_hbm.at[pl.ds(c * M_T + base, RPT)])


@functools.cache
def _cnt_call():
    return pl.kernel(
        _cnt_body,
        out_type=jax.ShapeDtypeStruct((NC * M_T, 16), jnp.float32),
        mesh=plsc.VectorSubcoreMesh(core_axis_name="c", subcore_axis_name="s",
                                    num_cores=NC, num_subcores=NS),
        scratch_types=[
            pltpu.VMEM((SS, EC), jnp.int32),
            pltpu.VMEM((EC, 16), jnp.float32),
            pltpu.VMEM((ZR, 16), jnp.float32),
            pltpu.VMEM_SHARED((M_T, 16), jnp.float32),
        ],
        compiler_params=pltpu.CompilerParams(use_tc_tiling_on_sc=False),
    )


def _prep_body(cf, wc, bc, qe, wq, bq, wm1, bm1,
               h_out, a_out, b_out, qv_out):
    acc = bc[...]
    for i in range(4):
        acc = acc + cf[:, i:i + 1] * wc[i:i + 1, :]
    h = _tc_gelu(acc)
    q = _tc_gelu(jnp.dot(qe[...], wq[...],
                         preferred_element_type=jnp.float32) + bq[...])
    a = jnp.dot(h, wm1[0:H, :], preferred_element_type=jnp.float32)
    c0 = jnp.dot(q, wm1[2 * H:3 * H, :],
                 preferred_element_type=jnp.float32) + bm1[...]
    b = jnp.dot(h, wm1[H:2 * H, :],
                preferred_element_type=jnp.float32) + c0
    h_out[...] = h
    a_out[0] = a[:, 0:HH]
    a_out[1] = a[:, HH:H]
    b_out[0] = b[:, 0:HH]
    b_out[1] = b[:, HH:H]

    @pl.when(pl.program_id(0) == 0)
    def _():
        qv_out[...] = q


def _node_body(final, h_ref, s_ref, cp_ref, wm2, bm2, wu1, bu1, wu2, bu2,
               gg, bb, *rest):
    h = h_ref[...]
    s64 = jnp.concatenate([s_ref[0], s_ref[1]], axis=-1)
    cnt = cp_ref[0, :, 0:1] + cp_ref[1, :, 0:1]
    inv = 1.0 / jnp.maximum(cnt, 1.0)
    occ = jnp.where(cnt > 0, 1.0, 0.0)
    agg = jnp.dot(s64 * inv, wm2[...],
                  preferred_element_type=jnp.float32) + occ * bm2[...]
    u = _tc_gelu(jnp.dot(h, wu1[0:H, :], preferred_element_type=jnp.float32)
                 + jnp.dot(agg, wu1[H:2 * H, :],
                           preferred_element_type=jnp.float32) + bu1[...])
    x = jnp.dot(u, wu2[...], preferred_element_type=jnp.float32) + bu2[...] + h
    mu = jnp.mean(x, axis=-1, keepdims=True)
    xc = x - mu
    var = jnp.mean(xc * xc, axis=-1, keepdims=True)
    hn = xc * lax.rsqrt(var + 1e-5) * gg[...] + bb[...]
    if final:
        ws1, bs1, ws2, bs2, out_ref = rest
        sc = jnp.dot(_tc_gelu(jnp.dot(hn, ws1[...],
                                      preferred_element_type=jnp.float32)
                              + bs1[...]),
                     ws2[...], preferred_element_type=jnp.float32) + bs2[...]
        out_ref[...] = sc
    else:
        qv, wm1n, bm1n, h_out, a_out, b_out = rest
        h_out[...] = hn
        a = jnp.dot(hn, wm1n[0:H, :], preferred_element_type=jnp.float32)
        cn = jnp.dot(qv[...], wm1n[2 * H:3 * H, :],
                     preferred_element_type=jnp.float32) + bm1n[...]
        b = jnp.dot(hn, wm1n[H:2 * H, :],
                    preferred_element_type=jnp.float32) + cn
        a_out[0] = a[:, 0:HH]
        a_out[1] = a[:, HH:H]
        b_out[0] = b[:, 0:HH]
        b_out[1] = b[:, HH:H]


def _full(shape):
    return pl.BlockSpec(shape, lambda i: tuple(0 for _ in shape))


_ROW = pl.BlockSpec((RB, H), lambda i: (i, 0))
_TBL = pl.BlockSpec((2, RB, HH), lambda i: (0, i, 0))

_prep_call = pl.pallas_call(
    _prep_body,
    grid=(M // RB,),
    in_specs=[
        pl.BlockSpec((RB, 4), lambda i: (i, 0)),
        _full((4, H)), _full((1, H)), _full((1, 768)), _full((768, H)),
        _full((1, H)), _full((3 * H, H)), _full((1, H)),
    ],
    out_specs=[_ROW, _TBL, _TBL, _full((1, H))],
    out_shape=[
        jax.ShapeDtypeStruct((M, H), jnp.float32),
        jax.ShapeDtypeStruct((2, M, HH), jnp.float32),
        jax.ShapeDtypeStruct((2, M, HH), jnp.float32),
        jax.ShapeDtypeStruct((1, H), jnp.float32),
    ],
    compiler_params=pltpu.CompilerParams(
        dimension_semantics=("arbitrary",)),
)

_node_mid_call = pl.pallas_call(
    functools.partial(_node_body, False),
    grid=(M // RB,),
    in_specs=[
        _ROW, _TBL, pl.BlockSpec((2, RB, 16), lambda i: (0, i, 0)),
        _full((H, H)), _full((1, H)), _full((2 * H, H)), _full((1, H)),
        _full((H, H)), _full((1, H)), _full((1, H)), _full((1, H)),
        _full((1, H)), _full((3 * H, H)), _full((1, H)),
    ],
    out_specs=[_ROW, _TBL, _TBL],
    out_shape=[
        jax.ShapeDtypeStruct((M, H), jnp.float32),
        jax.ShapeDtypeStruct((2, M, HH), jnp.float32),
        jax.ShapeDtypeStruct((2, M, HH), jnp.float32),
    ],
    compiler_params=pltpu.CompilerParams(
        dimension_semantics=("arbitrary",)),
)

_node_fin_call = pl.pallas_call(
    functools.partial(_node_body, True),
    grid=(M // RB,),
    in_specs=[
        _ROW, _TBL, pl.BlockSpec((2, RB, 16), lambda i: (0, i, 0)),
        _full((H, H)), _full((1, H)), _full((2 * H, H)), _full((1, H)),
        _full((H, H)), _full((1, H)), _full((1, H)), _full((1, H)),
        _full((H, HH)), _full((1, HH)), _full((HH, 1)), _full((1, 1)),
    ],
    out_specs=[pl.BlockSpec((RB, 1), lambda i: (i, 0))],
    out_shape=[jax.ShapeDtypeStruct((M, 1), jnp.float32)],
    compiler_params=pltpu.CompilerParams(
        dimension_semantics=("arbitrary",)),
)


def kernel(cell_features, edge_index, q_emb, W_cell, b_cell, W_q, b_q,
           l0_Wm1, l0_bm1, l0_Wm2, l0_bm2, l0_Wu1, l0_bu1, l0_Wu2, l0_bu2,
           l0_g, l0_b, l1_Wm1, l1_bm1, l1_Wm2, l1_bm2, l1_Wu1, l1_bu1,
           l1_Wu2, l1_bu2, l1_g, l1_b, W_s1, b_s1, W_s2, b_s2):
    r2 = lambda v: v.reshape(1, -1)
    src = edge_index[0]
    dst = edge_index[1]
    pad = jnp.zeros((NR_PAD * EC - E,), jnp.int32)
    offs = jnp.array([[0], [M]], jnp.int32)
    srcr = (jnp.concatenate([src, pad])[None, :] + offs).reshape(
        NC * NR_PAD, EC)
    dstr = (jnp.concatenate([dst, pad])[None, :] + offs).reshape(
        NC * NR_PAD, EC)

    h0, A0, B0, qv = _prep_call(
        cell_features, W_cell, r2(b_cell), r2(q_emb), W_q, r2(b_q),
        l0_Wm1, r2(l0_bm1))
    cntP = _cnt_call()(dstr).reshape(NC, M_T, 16)

    S0 = _edge_call()(A0.reshape(NC * M, HH), B0.reshape(NC * M, HH),
                      srcr, dstr).reshape(NC, M_T, HH)
    h1, A1, B1 = _node_mid_call(
        h0, S0, cntP, l0_Wm2, r2(l0_bm2), l0_Wu1, r2(l0_bu1), l0_Wu2,
        r2(l0_bu2), r2(l0_g), r2(l0_b), qv, l1_Wm1, r2(l1_bm1))

    S1 = _edge_call()(A1.reshape(NC * M, HH), B1.reshape(NC * M, HH),
                      srcr, dstr).reshape(NC, M_T, HH)
    scores = _node_fin_call(
        h1, S1, cntP, l1_Wm2, r2(l1_bm2), l1_Wu1, r2(l1_bu1), l1_Wu2,
        r2(l1_bu2), r2(l1_g), r2(l1_b), W_s1, r2(b_s1), W_s2, r2(b_s2))[0]
    return scores.reshape(M)


# speedup vs baseline: 4.3288x; 4.3288x over previous
import functools

import jax
import jax.numpy as jnp
from jax import lax
from jax.experimental import pallas as pl
from jax.experimental.pallas import tpu as pltpu
from jax.experimental.pallas import tpu_sc as plsc

M = 50000
E = 800000
H = 64
HH = 32
NC = 2
NS = 16
EC = 128
NR = E // EC
G8 = NR // 8
SS = 16
NR_PAD = 6304
M_T = 50048
RPT = M_T // NS
ZR = 136
RB = 2000


def _tc_gelu(x):
    return 0.5 * x * (1.0 + lax.erf(x * 0.7071067811865476))


def _edge_body(a_hbm, b_hbm, src_hbm, dsta_hbm, out_hbm,
               sidx, didxa, didxr, bufa, bufb, bufg, zb, tbl,
               sema0, sema1, semb0, semb1):
    c = lax.axis_index("c")
    s = lax.axis_index("s")
    sems = ((sema0, semb0), (sema1, semb1))
    zero16 = jnp.zeros((16,), jnp.float32)

    @pl.loop(0, ZR)
    def _zfill(r):
        zb[r, pl.ds(0, 16)] = zero16
        zb[r, pl.ds(16, 16)] = zero16

    base = s * RPT

    @pl.loop(0, RPT // ZR)
    def _zcopy(k):
        pltpu.sync_copy(zb, tbl.at[pl.ds(base + k * ZR, ZR)])

    plsc.subcore_barrier()

    start = 8 * ((G8 * s) // NS)
    end = jnp.where(s == NS - 1, NR, 8 * ((G8 * (s + 1)) // NS))
    cbase = c * NR_PAD

    def _issue(j, b):
        pltpu.async_copy(a_hbm.at[sidx.at[j]], bufa.at[b], sems[b][0])
        pltpu.async_copy(b_hbm.at[didxa.at[j]], bufb.at[b], sems[b][1])

    def _wait(j, b):
        pltpu.make_async_copy(a_hbm.at[sidx.at[j]], bufa.at[b],
                              sems[b][0]).wait()
        pltpu.make_async_copy(b_hbm.at[didxa.at[j]], bufb.at[b],
                              sems[b][1]).wait()

    def _compute(b):
        slots = [(rr, pl.ds(16 * h, 16)) for rr in range(4) for h in (0, 1)]

        @pl.loop(0, EC, step=4)
        def _rows(r):
            xs = [bufa[b, r + rr, sl] + bufb[b, r + rr, sl]
                  for rr, sl in slots]
            x2 = [x * x for x in xs]
            us = [-0.07135531702265918 * t for t in x2]
            us = [t - 1.5957691216057308 for t in us]
            ar = [x * t for x, t in zip(xs, us)]
            es = [jnp.exp(t) for t in ar]
            ds = [1.0 + t for t in es]
            rs = [x / d for x, d in zip(xs, ds)]
            for (rr, sl), v in zip(slots, rs):
                bufg[r + rr, sl] = v

    @pl.loop(start, end, step=SS)
    def _outer(g):
        g = pl.multiple_of(g, 8)
        m = jnp.minimum(end - g, SS)
        pltpu.sync_copy(src_hbm.at[pl.ds(cbase + g, SS)], sidx)
        pltpu.sync_copy(dsta_hbm.at[pl.ds(cbase + g, SS)], didxa)
        pltpu.sync_copy(dsta_hbm.at[pl.ds(g, SS)], didxr)
        _issue(0, 0)

        @pl.loop(0, (m + 1) // 2)
        def _pairs(p):
            j0 = 2 * p
            j1 = j0 + 1
            _wait(j0, 0)

            @pl.when(j1 < m)
            def _():
                _issue(j1, 1)

            _compute(0)
            pltpu.sync_copy(bufg, tbl.at[didxr.at[j0]], add=True)

            @pl.when(j1 < m)
            def _():
                _wait(j1, 1)

                @pl.when(j1 + 1 < m)
                def _():
                    _issue(j1 + 1, 0)

                _compute(1)
                pltpu.sync_copy(bufg, tbl.at[didxr.at[j1]], add=True)

    plsc.subcore_barrier()
    pltpu.sync_copy(tbl.at[pl.ds(base, RPT)],
                    out_hbm.at[pl.ds(c * M_T + base, RPT)])


@functools.cache
def _edge_call():
    return pl.kernel(
        _edge_body,
        out_type=jax.ShapeDtypeStruct((NC * M_T, HH), jnp.float32),
        mesh=plsc.VectorSubcoreMesh(core_axis_name="c", subcore_axis_name="s",
                                    num_cores=NC, num_subcores=NS),
        scratch_types=[
            pltpu.VMEM((SS, EC), jnp.int32),
            pltpu.VMEM((SS, EC), jnp.int32),
            pltpu.VMEM((SS, EC), jnp.int32),
            pltpu.VMEM((2, EC, HH), jnp.float32),
            pltpu.VMEM((2, EC, HH), jnp.float32),
            pltpu.VMEM((EC, HH), jnp.float32),
            pltpu.VMEM((ZR, HH), jnp.float32),
            pltpu.VMEM_SHARED((M_T, HH), jnp.float32),
            pltpu.SemaphoreType.DMA,
            pltpu.SemaphoreType.DMA,
            pltpu.SemaphoreType.DMA,
            pltpu.SemaphoreType.DMA,
        ],
        compiler_params=pltpu.CompilerParams(use_tc_tiling_on_sc=False),
    )


def _cnt_body(dsta_hbm, out_hbm, didx, ones, zb, ctbl):
    c = lax.axis_index("c")
    s = lax.axis_index("s")
    zero16 = jnp.zeros((16,), jnp.float32)
    onev = jnp.where(lax.broadcasted_iota(jnp.int32, (16,), 0) == 0,
                     1.0, 0.0).astype(jnp.float32)

    @pl.loop(0, EC)
    def _ofill(r):
        ones[r, pl.ds(0, 16)] = onev

    @pl.loop(0, ZR)
    def _zfill(r):
        zb[r, pl.ds(0, 16)] = zero16

    base = s * RPT

    @pl.loop(0, RPT // ZR)
    def _zcopy(k):
        pltpu.sync_copy(zb, ctbl.at[pl.ds(base + k * ZR, ZR)])

    plsc.subcore_barrier()

    w = c * NS + s
    nw = NC * NS
    start = 8 * ((G8 * w) // nw)
    end = jnp.where(w == nw - 1, NR, 8 * ((G8 * (w + 1)) // nw))

    @pl.loop(start, end, step=SS)
    def _outer(g):
        g = pl.multiple_of(g, 8)
        m = jnp.minimum(end - g, SS)
        pltpu.sync_copy(dsta_hbm.at[pl.ds(g, SS)], didx)

        @pl.loop(0, m)
        def _inner(j):
            pltpu.sync_copy(ones, ctbl.at[didx.at[j]], add=True)

    plsc.subcore_barrier()
    pltpu.sync_copy(ctbl.at[pl.ds(base, RPT)],
                    out_hbm.at[pl.ds(c * M_T + base, RPT)])


@functools.cache
def _cnt_call():
    return pl.kernel(
        _cnt_body,
        out_type=jax.ShapeDtypeStruct((NC * M_T, 16), jnp.float32),
        mesh=plsc.VectorSubcoreMesh(core_axis_name="c", subcore_axis_name="s",
                                    num_cores=NC, num_subcores=NS),
        scratch_types=[
            pltpu.VMEM((SS, EC), jnp.int32),
            pltpu.VMEM((EC, 16), jnp.float32),
            pltpu.VMEM((ZR, 16), jnp.float32),
            pltpu.VMEM_SHARED((M_T, 16), jnp.float32),
        ],
        compiler_params=pltpu.CompilerParams(use_tc_tiling_on_sc=False),
    )


def _prep_body(cf, wc, bc, qe, wq, bq, wm1, bm1,
               h_out, a_out, b_out, qv_out):
    acc = bc[...]
    for i in range(4):
        acc = acc + cf[:, i:i + 1] * wc[i:i + 1, :]
    h = _tc_gelu(acc)
    q = _tc_gelu(jnp.dot(qe[...], wq[...],
                         preferred_element_type=jnp.float32) + bq[...])
    a = jnp.dot(h, wm1[0:H, :], preferred_element_type=jnp.float32)
    c0 = jnp.dot(q, wm1[2 * H:3 * H, :],
                 preferred_element_type=jnp.float32) + bm1[...]
    b = jnp.dot(h, wm1[H:2 * H, :],
                preferred_element_type=jnp.float32) + c0
    h_out[...] = h
    a_out[0] = a[:, 0:HH]
    a_out[1] = a[:, HH:H]
    b_out[0] = b[:, 0:HH]
    b_out[1] = b[:, HH:H]

    @pl.when(pl.program_id(0) == 0)
    def _():
        qv_out[...] = q


def _node_body(final, h_ref, s_ref, cp_ref, wm2, bm2, wu1, bu1, wu2, bu2,
               gg, bb, *rest):
    h = h_ref[...]
    s64 = jnp.concatenate([s_ref[0], s_ref[1]], axis=-1)
    cnt = cp_ref[0, :, 0:1] + cp_ref[1, :, 0:1]
    inv = 1.0 / jnp.maximum(cnt, 1.0)
    occ = jnp.where(cnt > 0, 1.0, 0.0)
    agg = jnp.dot(s64 * inv, wm2[...],
                  preferred_element_type=jnp.float32) + occ * bm2[...]
    u = _tc_gelu(jnp.dot(h, wu1[0:H, :], preferred_element_type=jnp.float32)
                 + jnp.dot(agg, wu1[H:2 * H, :],
                           preferred_element_type=jnp.float32) + bu1[...])
    x = jnp.dot(u, wu2[...], preferred_element_type=jnp.float32) + bu2[...] + h
    mu = jnp.mean(x, axis=-1, keepdims=True)
    xc = x - mu
    var = jnp.mean(xc * xc, axis=-1, keepdims=True)
    hn = xc * lax.rsqrt(var + 1e-5) * gg[...] + bb[...]
    if final:
        ws1, bs1, ws2, bs2, out_ref = rest
        sc = jnp.dot(_tc_gelu(jnp.dot(hn, ws1[...],
                                      preferred_element_type=jnp.float32)
                              + bs1[...]),
                     ws2[...], preferred_element_type=jnp.float32) + bs2[...]
        out_ref[...] = sc
    else:
        qv, wm1n, bm1n, h_out, a_out, b_out = rest
        h_out[...] = hn
        a = jnp.dot(hn, wm1n[0:H, :], preferred_element_type=jnp.float32)
        cn = jnp.dot(qv[...], wm1n[2 * H:3 * H, :],
                     preferred_element_type=jnp.float32) + bm1n[...]
        b = jnp.dot(hn, wm1n[H:2 * H, :],
                    preferred_element_type=jnp.float32) + cn
        a_out[0] = a[:, 0:HH]
        a_out[1] = a[:, HH:H]
        b_out[0] = b[:, 0:HH]
        b_out[1] = b[:, HH:H]


def _full(shape):
    return pl.BlockSpec(shape, lambda i: tuple(0 for _ in shape))


_ROW = pl.BlockSpec((RB, H), lambda i: (i, 0))
_TBL = pl.BlockSpec((2, RB, HH), lambda i: (0, i, 0))

_prep_call = pl.pallas_call(
    _prep_body,
    grid=(M // RB,),
    in_specs=[
        pl.BlockSpec((RB, 4), lambda i: (i, 0)),
        _full((4, H)), _full((1, H)), _full((1, 768)), _full((768, H)),
        _full((1, H)), _full((3 * H, H)), _full((1, H)),
    ],
    out_specs=[_ROW, _TBL, _TBL, _full((1, H))],
    out_shape=[
        jax.ShapeDtypeStruct((M, H), jnp.float32),
        jax.ShapeDtypeStruct((2, M, HH), jnp.float32),
        jax.ShapeDtypeStruct((2, M, HH), jnp.float32),
        jax.ShapeDtypeStruct((1, H), jnp.float32),
    ],
    compiler_params=pltpu.CompilerParams(
        dimension_semantics=("arbitrary",)),
)

_node_mid_call = pl.pallas_call(
    functools.partial(_node_body, False),
    grid=(M // RB,),
    in_specs=[
        _ROW, _TBL, pl.BlockSpec((2, RB, 16), lambda i: (0, i, 0)),
        _full((H, H)), _full((1, H)), _full((2 * H, H)), _full((1, H)),
        _full((H, H)), _full((1, H)), _full((1, H)), _full((1, H)),
        _full((1, H)), _full((3 * H, H)), _full((1, H)),
    ],
    out_specs=[_ROW, _TBL, _TBL],
    out_shape=[
        jax.ShapeDtypeStruct((M, H), jnp.float32),
        jax.ShapeDtypeStruct((2, M, HH), jnp.float32),
        jax.ShapeDtypeStruct((2, M, HH), jnp.float32),
    ],
    compiler_params=pltpu.CompilerParams(
        dimension_semantics=("arbitrary",)),
)

_node_fin_call = pl.pallas_call(
    functools.partial(_node_body, True),
    grid=(M // RB,),
    in_specs=[
        _ROW, _TBL, pl.BlockSpec((2, RB, 16), lambda i: (0, i, 0)),
        _full((H, H)), _full((1, H)), _full((2 * H, H)), _full((1, H)),
        _full((H, H)), _full((1, H)), _full((1, H)), _full((1, H)),
        _full((H, HH)), _full((1, HH)), _full((HH, 1)), _full((1, 1)),
    ],
    out_specs=[pl.BlockSpec((RB, 1), lambda i: (i, 0))],
    out_shape=[jax.ShapeDtypeStruct((M, 1), jnp.float32)],
    compiler_params=pltpu.CompilerParams(
        dimension_semantics=("arbitrary",)),
)


def kernel(cell_features, edge_index, q_emb, W_cell, b_cell, W_q, b_q,
           l0_Wm1, l0_bm1, l0_Wm2, l0_bm2, l0_Wu1, l0_bu1, l0_Wu2, l0_bu2,
           l0_g, l0_b, l1_Wm1, l1_bm1, l1_Wm2, l1_bm2, l1_Wu1, l1_bu1,
           l1_Wu2, l1_bu2, l1_g, l1_b, W_s1, b_s1, W_s2, b_s2):
    r2 = lambda v: v.reshape(1, -1)
    src = edge_index[0]
    dst = edge_index[1]
    pad = jnp.zeros((NR_PAD * EC - E,), jnp.int32)
    offs = jnp.array([[0], [M]], jnp.int32)
    srcr = (jnp.concatenate([src, pad])[None, :] + offs).reshape(
        NC * NR_PAD, EC)
    dstr = (jnp.concatenate([dst, pad])[None, :] + offs).reshape(
        NC * NR_PAD, EC)

    h0, A0, B0, qv = _prep_call(
        cell_features, W_cell, r2(b_cell), r2(q_emb), W_q, r2(b_q),
        l0_Wm1, r2(l0_bm1))
    cntP = _cnt_call()(dstr).reshape(NC, M_T, 16)

    S0 = _edge_call()(A0.reshape(NC * M, HH), B0.reshape(NC * M, HH),
                      srcr, dstr).reshape(NC, M_T, HH)
    h1, A1, B1 = _node_mid_call(
        h0, S0, cntP, l0_Wm2, r2(l0_bm2), l0_Wu1, r2(l0_bu1), l0_Wu2,
        r2(l0_bu2), r2(l0_g), r2(l0_b), qv, l1_Wm1, r2(l1_bm1))

    S1 = _edge_call()(A1.reshape(NC * M, HH), B1.reshape(NC * M, HH),
                      srcr, dstr).reshape(NC, M_T, HH)
    scores = _node_fin_call(
        h1, S1, cntP, l1_Wm2, r2(l1_bm2), l1_Wu1, r2(l1_bu1), l1_Wu2,
        r2(l1_bu2), r2(l1_g), r2(l1_b), W_s1, r2(b_s1), W_s2, r2(b_s2))[0]
    return scores.reshape(M)
